# baseline (speedup 1.0000x reference)
import jax
import jax.numpy as jnp
from jax.experimental import pallas as pl
from jax.experimental.pallas import tpu as pltpu

_N = 20000
_ROWS = 160
_COLS = 128
_N_PAD = _ROWS * _COLS
_SCORE_THRESH = 0.05
_NMS_THRESH = 0.5
_K = 100
_NEG = -1e9


def _combine(av, af, bv, bf):
    take = (av > bv) | ((av == bv) & (af < bf))
    return jnp.where(take, av, bv), jnp.where(take, af, bf)


def _argmax_pair(s, idx):
    vs = [s[k * 8:(k + 1) * 8] for k in range(_ROWS // 8)]
    fs = [idx[k * 8:(k + 1) * 8] for k in range(_ROWS // 8)]
    while len(vs) > 1:
        nv, nf = [], []
        for a in range(0, len(vs) - 1, 2):
            v, f = _combine(vs[a], fs[a], vs[a + 1], fs[a + 1])
            nv.append(v)
            nf.append(f)
        if len(vs) % 2:
            nv.append(vs[-1])
            nf.append(fs[-1])
        vs, fs = nv, nf
    v, f = vs[0], fs[0]
    for sh in (4, 2, 1):
        v, f = _combine(v, f, pltpu.roll(v, sh, 0), pltpu.roll(f, sh, 0))
    for sh in (64, 32, 16, 8, 4, 2, 1):
        v, f = _combine(v, f, pltpu.roll(v, sh, 1), pltpu.roll(f, sh, 1))
    return v, f


def _nms_body(x1_ref, y1_ref, x2_ref, y2_ref, s_ref, out_ref):
    x1 = x1_ref[...]
    y1 = y1_ref[...]
    x2 = x2_ref[...]
    y2 = y2_ref[...]
    scores = s_ref[...]
    s0 = jnp.where(scores > _SCORE_THRESH, scores, _NEG)
    area3 = (x2 - x1) * (y2 - y1) * (1.0 / 3.0)
    rid = jax.lax.broadcasted_iota(jnp.int32, (_ROWS, _COLS), 0)
    cid = jax.lax.broadcasted_iota(jnp.int32, (_ROWS, _COLS), 1)
    idx = rid * _COLS + cid
    lane = jax.lax.broadcasted_iota(jnp.int32, (1, _COLS), 1)
    ones = jnp.ones((_COLS, _COLS), jnp.float32)

    def body(i, s):
        mv, mf = _argmax_pair(s, idx)
        m = mv[0:1, :]
        sel = idx == mf[0:1, :]
        zero = jnp.zeros_like(s)
        tx1 = jnp.sum(jnp.where(sel, x1, zero), axis=0, keepdims=True)
        ty1 = jnp.sum(jnp.where(sel, y1, zero), axis=0, keepdims=True)
        tx2 = jnp.sum(jnp.where(sel, x2, zero), axis=0, keepdims=True)
        ty2 = jnp.sum(jnp.where(sel, y2, zero), axis=0, keepdims=True)
        t = jnp.concatenate([tx1, ty1, tx2, ty2], axis=0)
        b = jax.lax.dot_general(
            t, ones, (((1,), (0,)), ((), ())),
            preferred_element_type=jnp.float32,
        )
        bx1 = b[0:1, :]
        by1 = b[1:2, :]
        bx2 = b[2:3, :]
        by2 = b[3:4, :]
        barea3 = (bx2 - bx1) * (by2 - by1) * (1.0 / 3.0)
        valid = m > _NEG / 2.0

        iw = jnp.maximum(jnp.minimum(bx2, x2) - jnp.maximum(bx1, x1), 0.0)
        ih = jnp.maximum(jnp.minimum(by2, y2) - jnp.maximum(by1, y1), 0.0)
        inter = iw * ih
        suppress = inter > area3 + (barea3 + 1e-9 / 3.0)
        s = jnp.where(suppress, _NEG, s)

        row = (
            jnp.where(lane == 0, bx1, 0.0)
            + jnp.where(lane == 1, by1, 0.0)
            + jnp.where(lane == 2, bx2, 0.0)
            + jnp.where(lane == 3, by2, 0.0)
            + jnp.where(lane == 4, m, 0.0)
        )
        out_ref[pl.ds(i, 1), :] = jnp.where(valid, row, 0.0)
        return s

    jax.lax.fori_loop(0, _K, body, s0, unroll=False)


def kernel(boxes, scores):
    pad = _N_PAD - _N
    x1 = jnp.pad(boxes[:, 0], (0, pad)).reshape(_ROWS, _COLS)
    y1 = jnp.pad(boxes[:, 1], (0, pad)).reshape(_ROWS, _COLS)
    x2 = jnp.pad(boxes[:, 2], (0, pad)).reshape(_ROWS, _COLS)
    y2 = jnp.pad(boxes[:, 3], (0, pad)).reshape(_ROWS, _COLS)
    s = jnp.pad(scores, (0, pad), constant_values=-1.0).reshape(_ROWS, _COLS)

    out = pl.pallas_call(
        _nms_body,
        out_shape=jax.ShapeDtypeStruct((_K, _COLS), jnp.float32),
        in_specs=[pl.BlockSpec(memory_space=pltpu.VMEM)] * 5,
        out_specs=pl.BlockSpec(memory_space=pltpu.VMEM),
    )(x1, y1, x2, y2, s)
    return out[:, :5]


# speedup vs baseline: 1.1946x; 1.1946x over previous
import jax
import jax.numpy as jnp
from jax.experimental import pallas as pl
from jax.experimental.pallas import tpu as pltpu

_N = 20000
_ROWS = 160
_COLS = 128
_N_PAD = _ROWS * _COLS
_SCORE_THRESH = 0.05
_NMS_THRESH = 0.5
_K = 100
_NEG = -1e9
_BIG = 1.0e9


def _nms_body(x1_ref, y1_ref, x2_ref, y2_ref, s_ref, out_ref):
    x1 = x1_ref[...]
    y1 = y1_ref[...]
    x2 = x2_ref[...]
    y2 = y2_ref[...]
    scores = s_ref[...]
    s0 = jnp.where(scores > _SCORE_THRESH, scores, _NEG)
    area3 = (x2 - x1) * (y2 - y1) * (1.0 / 3.0)
    rid = jax.lax.broadcasted_iota(jnp.int32, (_ROWS, _COLS), 0)
    cid = jax.lax.broadcasted_iota(jnp.int32, (_ROWS, _COLS), 1)
    fidx = (rid * _COLS + cid).astype(jnp.float32)
    lane = jax.lax.broadcasted_iota(jnp.int32, (1, _COLS), 1)
    eye_r = jax.lax.broadcasted_iota(jnp.int32, (_COLS, _COLS), 0)
    eye_c = jax.lax.broadcasted_iota(jnp.int32, (_COLS, _COLS), 1)
    eye = (eye_r == eye_c).astype(jnp.float32)
    ones_col = jnp.ones((_COLS, 1), jnp.float32)
    ones_mat = jnp.ones((_COLS, _COLS), jnp.float32)
    nchunk = _ROWS // 8

    def body(i, s):
        ps = [
            (
                s[k * 8:(k + 1) * 8],
                fidx[k * 8:(k + 1) * 8],
                x1[k * 8:(k + 1) * 8],
                y1[k * 8:(k + 1) * 8],
                x2[k * 8:(k + 1) * 8],
                y2[k * 8:(k + 1) * 8],
            )
            for k in range(nchunk)
        ]
        while len(ps) > 1:
            nxt = []
            for a in range(0, len(ps) - 1, 2):
                pa, pb = ps[a], ps[a + 1]
                take = (pa[0] > pb[0]) | ((pa[0] == pb[0]) & (pa[1] < pb[1]))
                nxt.append(tuple(jnp.where(take, u, v) for u, v in zip(pa, pb)))
            if len(ps) % 2:
                nxt.append(ps[-1])
            ps = nxt
        w8 = ps[0]
        t4 = [u[0:4] for u in w8]
        b4 = [u[4:8] for u in w8]
        take = (t4[0] > b4[0]) | ((t4[0] == b4[0]) & (t4[1] < b4[1]))
        w4 = [jnp.where(take, u, v) for u, v in zip(t4, b4)]
        t2 = [u[0:2] for u in w4]
        b2 = [u[2:4] for u in w4]
        take = (t2[0] > b2[0]) | ((t2[0] == b2[0]) & (t2[1] < b2[1]))
        w2 = [jnp.where(take, u, v) for u, v in zip(t2, b2)]
        t1 = [u[0:1] for u in w2]
        b1 = [u[1:2] for u in w2]
        take = (t1[0] > b1[0]) | ((t1[0] == b1[0]) & (t1[1] < b1[1]))
        w1 = [jnp.where(take, u, v) for u, v in zip(t1, b1)]
        colv, colf, colx1, coly1, colx2, coly2 = w1

        dv = jnp.broadcast_to(colv, (_COLS, _COLS)) * eye
        df = jnp.broadcast_to(colf, (_COLS, _COLS)) * eye
        col_v = jax.lax.dot_general(
            dv, ones_col, (((1,), (0,)), ((), ())),
            preferred_element_type=jnp.float32,
            precision=jax.lax.Precision.HIGHEST,
        )
        col_f = jax.lax.dot_general(
            df, ones_col, (((1,), (0,)), ((), ())),
            preferred_element_type=jnp.float32,
            precision=jax.lax.Precision.HIGHEST,
        )
        m = jnp.max(col_v, axis=0, keepdims=True)
        fwin = jnp.min(
            jnp.where(col_v == m, col_f, _BIG), axis=0, keepdims=True
        )
        valid = m > _NEG / 2.0

        lane_hot = colf == fwin
        stack4 = jnp.concatenate([colx1, coly1, colx2, coly2], axis=0)
        t = jnp.where(lane_hot, stack4, 0.0)
        b = jax.lax.dot_general(
            t, ones_mat, (((1,), (0,)), ((), ())),
            preferred_element_type=jnp.float32,
            precision=jax.lax.Precision.HIGHEST,
        )
        bx1 = b[0:1, :]
        by1 = b[1:2, :]
        bx2 = b[2:3, :]
        by2 = b[3:4, :]
        barea3 = (bx2 - bx1) * (by2 - by1) * (1.0 / 3.0)

        iw = jnp.maximum(jnp.minimum(bx2, x2) - jnp.maximum(bx1, x1), 0.0)
        ih = jnp.maximum(jnp.minimum(by2, y2) - jnp.maximum(by1, y1), 0.0)
        inter = iw * ih
        suppress = inter > area3 + (barea3 + 1e-9 / 3.0)
        s = jnp.where(suppress, _NEG, s)

        row = (
            jnp.where(lane == 0, bx1, 0.0)
            + jnp.where(lane == 1, by1, 0.0)
            + jnp.where(lane == 2, bx2, 0.0)
            + jnp.where(lane == 3, by2, 0.0)
            + jnp.where(lane == 4, m, 0.0)
        )
        out_ref[pl.ds(i, 1), :] = jnp.where(valid, row, 0.0)
        return s

    jax.lax.fori_loop(0, _K, body, s0, unroll=False)


def kernel(boxes, scores):
    pad = _N_PAD - _N
    x1 = jnp.pad(boxes[:, 0], (0, pad)).reshape(_ROWS, _COLS)
    y1 = jnp.pad(boxes[:, 1], (0, pad)).reshape(_ROWS, _COLS)
    x2 = jnp.pad(boxes[:, 2], (0, pad)).reshape(_ROWS, _COLS)
    y2 = jnp.pad(boxes[:, 3], (0, pad)).reshape(_ROWS, _COLS)
    s = jnp.pad(scores, (0, pad), constant_values=-1.0).reshape(_ROWS, _COLS)

    out = pl.pallas_call(
        _nms_body,
        out_shape=jax.ShapeDtypeStruct((_K, _COLS), jnp.float32),
        in_specs=[pl.BlockSpec(memory_space=pltpu.VMEM)] * 5,
        out_specs=pl.BlockSpec(memory_space=pltpu.VMEM),
    )(x1, y1, x2, y2, s)
    return out[:, :5]


# speedup vs baseline: 1.9721x; 1.6508x over previous
import jax
import jax.numpy as jnp
from jax.experimental import pallas as pl
from jax.experimental.pallas import tpu as pltpu

_N = 20000
_ROWS = 160
_COLS = 128
_N_PAD = _ROWS * _COLS
_SCORE_THRESH = 0.05
_NMS_THRESH = 0.5
_K = 100
_NEG = -1e9


def _nms_body(x1_ref, y1_ref, x2_ref, y2_ref, s_ref, out_ref):
    x1 = x1_ref[...]
    y1 = y1_ref[...]
    x2 = x2_ref[...]
    y2 = y2_ref[...]
    scores = s_ref[...]
    s0 = jnp.where(scores > _SCORE_THRESH, scores, _NEG)
    area3 = (x2 - x1) * (y2 - y1) * (1.0 / 3.0)
    rowid = jax.lax.broadcasted_iota(jnp.int32, (_ROWS, _COLS), 0)
    lane = jax.lax.broadcasted_iota(jnp.int32, (1, _COLS), 1)
    ones_mat = jnp.ones((_COLS, _COLS), jnp.float32)
    nchunk = _ROWS // 8

    def body(i, s):
        ps = [
            (
                s[k * 8:(k + 1) * 8],
                rowid[k * 8:(k + 1) * 8],
                x1[k * 8:(k + 1) * 8],
                y1[k * 8:(k + 1) * 8],
                x2[k * 8:(k + 1) * 8],
                y2[k * 8:(k + 1) * 8],
            )
            for k in range(nchunk)
        ]
        while len(ps) > 1:
            nxt = []
            for a in range(0, len(ps) - 1, 2):
                pa, pb = ps[a], ps[a + 1]
                take = (pa[0] > pb[0]) | ((pa[0] == pb[0]) & (pa[1] < pb[1]))
                nxt.append(tuple(jnp.where(take, u, v) for u, v in zip(pa, pb)))
            if len(ps) % 2:
                nxt.append(ps[-1])
            ps = nxt
        w8 = ps[0]
        t4 = [u[0:4] for u in w8]
        b4 = [u[4:8] for u in w8]
        take = (t4[0] > b4[0]) | ((t4[0] == b4[0]) & (t4[1] < b4[1]))
        w4 = [jnp.where(take, u, v) for u, v in zip(t4, b4)]
        t2 = [u[0:2] for u in w4]
        b2 = [u[2:4] for u in w4]
        take = (t2[0] > b2[0]) | ((t2[0] == b2[0]) & (t2[1] < b2[1]))
        w2 = [jnp.where(take, u, v) for u, v in zip(t2, b2)]
        t1 = [u[0:1] for u in w2]
        b1 = [u[1:2] for u in w2]
        take = (t1[0] > b1[0]) | ((t1[0] == b1[0]) & (t1[1] < b1[1]))
        w1 = [jnp.where(take, u, v) for u, v in zip(t1, b1)]
        colv, _, colx1, coly1, colx2, coly2 = w1

        c = jnp.argmax(colv, axis=1).reshape(1, 1)
        lane_hot = lane == c

        stack = jnp.concatenate(
            [colx1, coly1, colx2, coly2, colv], axis=0
        )
        t = jnp.where(lane_hot, stack, 0.0)
        b = jax.lax.dot_general(
            t, ones_mat, (((1,), (0,)), ((), ())),
            preferred_element_type=jnp.float32,
            precision=jax.lax.Precision.HIGHEST,
        )
        bx1 = b[0:1, :]
        by1 = b[1:2, :]
        bx2 = b[2:3, :]
        by2 = b[3:4, :]
        m = b[4:5, :]
        barea3 = (bx2 - bx1) * (by2 - by1) * (1.0 / 3.0)
        valid = m > _NEG / 2.0

        iw = jnp.maximum(jnp.minimum(bx2, x2) - jnp.maximum(bx1, x1), 0.0)
        ih = jnp.maximum(jnp.minimum(by2, y2) - jnp.maximum(by1, y1), 0.0)
        inter = iw * ih
        suppress = inter > area3 + (barea3 + 1e-9 / 3.0)
        s = jnp.where(suppress, _NEG, s)

        row = (
            jnp.where(lane == 0, bx1, 0.0)
            + jnp.where(lane == 1, by1, 0.0)
            + jnp.where(lane == 2, bx2, 0.0)
            + jnp.where(lane == 3, by2, 0.0)
            + jnp.where(lane == 4, m, 0.0)
        )
        out_ref[pl.ds(i, 1), :] = jnp.where(valid, row, 0.0)
        return s

    jax.lax.fori_loop(0, _K, body, s0, unroll=False)


def _to_layout(v, fill):
    v = jnp.pad(v, (0, _N_PAD - _N), constant_values=fill)
    return v.reshape(_COLS, _ROWS).T[:, ::-1]


def kernel(boxes, scores):
    x1 = _to_layout(boxes[:, 0], 0.0)
    y1 = _to_layout(boxes[:, 1], 0.0)
    x2 = _to_layout(boxes[:, 2], 0.0)
    y2 = _to_layout(boxes[:, 3], 0.0)
    s = _to_layout(scores, -1.0)

    out = pl.pallas_call(
        _nms_body,
        out_shape=jax.ShapeDtypeStruct((_K, _COLS), jnp.float32),
        in_specs=[pl.BlockSpec(memory_space=pltpu.VMEM)] * 5,
        out_specs=pl.BlockSpec(memory_space=pltpu.VMEM),
    )(x1, y1, x2, y2, s)
    return out[:, :5]
